# Initial kernel scaffold; baseline (speedup 1.0000x reference)
import jax
import jax.numpy as jnp
from jax.experimental import pallas as pl


def kernel(node_feats, edge_index, edge_feats, W1, b1, W2, b2, W3, b3, root, bias):
    raise NotImplementedError("write your pallas kernel here")


# speedup vs baseline: 3.1400x; 3.1400x over previous
import functools

import jax
import jax.numpy as jnp
from jax import lax
from jax.experimental import pallas as pl
from jax.experimental.pallas import tpu as pltpu
from jax.experimental.pallas import tpu_sc as plsc

N_NODES = 10000
N_EDGES = 160000
NODE_IN = 128
EDGE_IN = 16
OUT_CH = 16
HIDDEN = 64

NUM_CORES = 2
NUM_SUBCORES = 16
NW = NUM_CORES * NUM_SUBCORES

NPAD = 12288
P_NODES = NPAD // NW

E_PH1 = 64000
E_PH2 = 96000

NEG_INF = float("-inf")


def _sc_mesh():
    return plsc.VectorSubcoreMesh(core_axis_name="c", subcore_axis_name="s")


def _wid():
    return lax.axis_index("s") * NUM_CORES + lax.axis_index("c")


G_CHUNK = 200


def _gather_body(eo, epw, nf_hbm, src_hbm, xj_hbm, idx_all, rows_v, sg0,
                 sg1, sw0, sw1):
    gsteps = epw // G_CHUNK
    wid = _wid()
    base = eo + wid * epw
    obase = wid * epw
    pltpu.sync_copy(src_hbm.at[pl.ds(base, epw)], idx_all)
    gs = (sg0, sg1)
    ws = (sw0, sw1)

    def g_args(j, b):
        return (nf_hbm.at[idx_all.at[pl.ds(j * G_CHUNK, G_CHUNK)]],
                rows_v.at[b], gs[b])

    def w_args(j, b):
        return (rows_v.at[b],
                xj_hbm.at[pl.ds(obase + j * G_CHUNK, G_CHUNK)], ws[b])

    pltpu.async_copy(*g_args(0, 0))
    for j in range(gsteps):
        b = j % 2
        if j + 1 < gsteps:
            if j >= 1:
                pltpu.make_async_copy(*w_args(j - 1, 1 - b)).wait()
            pltpu.async_copy(*g_args(j + 1, 1 - b))
        pltpu.make_async_copy(*g_args(j, b)).wait()
        pltpu.async_copy(*w_args(j, b))
    pltpu.make_async_copy(*w_args(gsteps - 2, (gsteps - 2) % 2)).wait()
    pltpu.make_async_copy(*w_args(gsteps - 1, (gsteps - 1) % 2)).wait()


def _gather_call(node_feats, src, eo, ne):
    epw = ne // NW
    k = functools.partial(
        pl.kernel,
        out_type=jax.ShapeDtypeStruct((ne, NODE_IN), jnp.float32),
        mesh=_sc_mesh(),
        compiler_params=pltpu.CompilerParams(needs_layout_passes=False),
        scratch_types=[
            pltpu.VMEM((epw,), jnp.int32),
            pltpu.VMEM((2, G_CHUNK, NODE_IN), jnp.float32),
            pltpu.SemaphoreType.DMA,
            pltpu.SemaphoreType.DMA,
            pltpu.SemaphoreType.DMA,
            pltpu.SemaphoreType.DMA,
        ],
    )(functools.partial(_gather_body, eo, epw))
    return k(node_feats, src)


BLK_E = 800


def _msg_body(ef_ref, xj_ref, w1_ref, b1_ref, w2_ref, b2_ref, w3q_ref,
              s2_ref, b3r_ref, out_ref):
    def mm(a, b):
        return lax.dot_general(a, b, (((1,), (0,)), ((), ())),
                               preferred_element_type=jnp.float32)

    h = jnp.maximum(mm(ef_ref[...], w1_ref[...]) + b1_ref[...], 0.0)
    h = jnp.maximum(mm(h, w2_ref[...]) + b2_ref[...], 0.0)
    xj = xj_ref[...].astype(jnp.bfloat16)
    g = mm(xj, w3q_ref[...])
    ht = jnp.tile(h, (1, OUT_CH))
    prod = (g * ht).astype(jnp.bfloat16)
    out_ref[...] = mm(prod, s2_ref[...]) + mm(xj, b3r_ref[...])


def _msg_call(ef_full, xj, W1, b1r, W2, b2r, W3q, S2, B3r, blk_off, ne):
    return pl.pallas_call(
        _msg_body,
        grid=(ne // BLK_E,),
        in_specs=[
            pl.BlockSpec((BLK_E, EDGE_IN), lambda i, o=blk_off: (i + o, 0)),
            pl.BlockSpec((BLK_E, NODE_IN), lambda i: (i, 0)),
            pl.BlockSpec((EDGE_IN, HIDDEN), lambda i: (0, 0)),
            pl.BlockSpec((1, HIDDEN), lambda i: (0, 0)),
            pl.BlockSpec((HIDDEN, HIDDEN), lambda i: (0, 0)),
            pl.BlockSpec((1, HIDDEN), lambda i: (0, 0)),
            pl.BlockSpec((NODE_IN, OUT_CH * HIDDEN), lambda i: (0, 0)),
            pl.BlockSpec((OUT_CH * HIDDEN, OUT_CH), lambda i: (0, 0)),
            pl.BlockSpec((NODE_IN, OUT_CH), lambda i: (0, 0)),
        ],
        out_specs=pl.BlockSpec((BLK_E, OUT_CH), lambda i: (i, 0)),
        out_shape=jax.ShapeDtypeStruct((ne, OUT_CH), jnp.float32),
        compiler_params=pltpu.CompilerParams(
            dimension_semantics=("arbitrary",)),
    )(ef_full, xj, W1, b1r, W2, b2r, W3q, S2, B3r)


ROOT_BLK = 2000


def _root_body(nf_ref, root_ref, bias_ref, out_ref):
    out_ref[...] = lax.dot_general(
        nf_ref[...], root_ref[...], (((1,), (0,)), ((), ())),
        preferred_element_type=jnp.float32) + bias_ref[...]


def _root_call(node_feats, root, biasr):
    return pl.pallas_call(
        _root_body,
        grid=(N_NODES // ROOT_BLK,),
        in_specs=[
            pl.BlockSpec((ROOT_BLK, NODE_IN), lambda i: (i, 0)),
            pl.BlockSpec((NODE_IN, OUT_CH), lambda i: (0, 0)),
            pl.BlockSpec((1, OUT_CH), lambda i: (0, 0)),
        ],
        out_specs=pl.BlockSpec((ROOT_BLK, OUT_CH), lambda i: (i, 0)),
        out_shape=jax.ShapeDtypeStruct((NPAD, OUT_CH), jnp.float32),
    )(node_feats, root, biasr)


NSPLIT = 8
CPT = 4
SLAB = 400
SUB = SLAB // 16


def _scatter_body(eo, es, nslab, has_init, msg_hbm, dst_hbm, *rest):
    if has_init:
        (pin_hbm, part_hbm, dst_v0, dst_v1, msg_v0, msg_v1, acc_v, tag_v,
         sd0, sd1, sm0, sm1) = rest
    else:
        pin_hbm = None
        (part_hbm, dst_v0, dst_v1, msg_v0, msg_v1, acc_v, tag_v,
         sd0, sd1, sm0, sm1) = rest
    w = _wid()
    split = w % NSPLIT
    grp = w // NSPLIT
    iota = lax.iota(jnp.int32, 16)
    iota16 = iota * 16
    neg = jnp.full((16,), NEG_INF, jnp.float32)
    dsts = (dst_v0, dst_v1)
    msgs = (msg_v0, msg_v1)
    sds = (sd0, sd1)
    sms = (sm0, sm1)

    def d_args(j, b):
        return (dst_hbm.at[pl.ds(eo + split * es + j * SLAB, SLAB)],
                dsts[b], sds[b])

    def m_args(j, b):
        return (msg_hbm.at[pl.ds((split * es + j * SLAB) * OUT_CH,
                                 SLAB * OUT_CH)], msgs[b], sms[b])

    pltpu.async_copy(*d_args(0, 0))
    pltpu.async_copy(*m_args(0, 0))
    pltpu.async_copy(*d_args(1, 1))
    pltpu.async_copy(*m_args(1, 1))

    if has_init:
        for ci in range(CPT):
            pltpu.sync_copy(
                pin_hbm.at[pl.ds((split * OUT_CH + grp * CPT + ci) * NPAD,
                                 NPAD)],
                acc_v.at[pl.ds(ci * NPAD, NPAD)])
    else:
        def init_step(t, carry):
            acc_v[pl.ds(t * 16, 16)] = neg
            return carry

        lax.fori_loop(0, CPT * NPAD // 16, init_step, 0)

    def pair_step(jj, carry):
        for b in range(2):
            j = jj * 2 + b
            pltpu.make_async_copy(*d_args(j, b)).wait()
            pltpu.make_async_copy(*m_args(j, b)).wait()

            def sub_step(k, c2):
                dstv = dsts[b][pl.ds(k * 16, 16)]
                plsc.store_scatter(tag_v, [dstv], iota)
                tag = plsc.load_gather(tag_v, [dstv])
                ndup = jnp.max(jnp.where(tag != iota, 1, 0))

                @pl.when(ndup == 0)
                def _fast():
                    for ci in range(CPT):
                        vidx = iota16 + (k * 16 * OUT_CH + grp * CPT + ci)
                        aidx = dstv + ci * NPAD
                        vals = plsc.load_gather(msgs[b], [vidx])
                        cur = plsc.load_gather(acc_v, [aidx])
                        plsc.store_scatter(acc_v, [aidx],
                                           jnp.maximum(cur, vals))

                @pl.when(ndup != 0)
                def _slow():
                    def rmw(mask_b):
                        failed = None
                        for ci in range(CPT):
                            vidx = iota16 + (k * 16 * OUT_CH + grp * CPT + ci)
                            aidx = dstv + ci * NPAD
                            vals = plsc.load_gather(msgs[b], [vidx],
                                                    mask=mask_b)
                            cur = plsc.load_gather(acc_v, [aidx], mask=mask_b)
                            new = jnp.maximum(cur, vals)
                            plsc.store_scatter(acc_v, [aidx], new,
                                               mask=mask_b)
                            cur2 = plsc.load_gather(acc_v, [aidx],
                                                    mask=mask_b)
                            f = jnp.logical_and(mask_b, cur2 < new)
                            failed = (f if failed is None
                                      else jnp.logical_or(failed, f))
                        return failed.astype(jnp.int32)

                    f0 = rmw(iota >= 0)

                    def w_cond(c):
                        return c[0] > 0

                    def w_body(c):
                        _, fm = c
                        f = rmw(fm != 0)
                        return jnp.max(f), f

                    lax.while_loop(w_cond, w_body, (jnp.max(f0), f0))

                return c2

            lax.fori_loop(0, SUB, sub_step, 0)

            @pl.when(jj * 2 + b + 2 < nslab)
            def _prefetch():
                pltpu.async_copy(*d_args(j + 2, b))
                pltpu.async_copy(*m_args(j + 2, b))

        return carry

    lax.fori_loop(0, nslab // 2, pair_step, 0)

    for ci in range(CPT):
        pltpu.sync_copy(
            acc_v.at[pl.ds(ci * NPAD, NPAD)],
            part_hbm.at[pl.ds((split * OUT_CH + grp * CPT + ci) * NPAD, NPAD)])


def _scatter_call(msg_flat, dst, eo, ne, part_in=None):
    es = ne // NSPLIT
    nslab = es // SLAB
    scratch = [
        pltpu.VMEM((SLAB,), jnp.int32),
        pltpu.VMEM((SLAB,), jnp.int32),
        pltpu.VMEM((SLAB * OUT_CH,), jnp.float32),
        pltpu.VMEM((SLAB * OUT_CH,), jnp.float32),
        pltpu.VMEM((CPT * NPAD,), jnp.float32),
        pltpu.VMEM((NPAD,), jnp.int32),
        pltpu.SemaphoreType.DMA,
        pltpu.SemaphoreType.DMA,
        pltpu.SemaphoreType.DMA,
        pltpu.SemaphoreType.DMA,
    ]
    has_init = part_in is not None
    k = functools.partial(
        pl.kernel,
        out_type=jax.ShapeDtypeStruct((NSPLIT * OUT_CH * NPAD,), jnp.float32),
        mesh=_sc_mesh(),
        compiler_params=pltpu.CompilerParams(needs_layout_passes=False),
        scratch_types=scratch,
    )(functools.partial(_scatter_body, eo, es, nslab, has_init))
    if has_init:
        return k(msg_flat, dst, part_in)
    return k(msg_flat, dst)


def _final_body(part_hbm, root_hbm, out_hbm, p_v, root_v, out_v):
    w = _wid()
    base = w * P_NODES
    pltpu.sync_copy(part_hbm.at[:, :, pl.ds(base, P_NODES)], p_v)
    pltpu.sync_copy(root_hbm.at[pl.ds(base * OUT_CH, P_NODES * OUT_CH)],
                    root_v)
    iota = lax.iota(jnp.int32, 16)
    iota16 = iota * 16

    def chunk_step(t, carry):
        for c in range(OUT_CH):
            m = p_v[0, c, pl.ds(t * 16, 16)]
            for s in range(1, NSPLIT):
                m = jnp.maximum(m, p_v[s, c, pl.ds(t * 16, 16)])
            fixed = jnp.where(m == NEG_INF, 0.0, m)
            ridx = iota16 + (t * 16 * OUT_CH + c)
            r = plsc.load_gather(root_v, [ridx])
            plsc.store_scatter(out_v, [ridx], fixed + r)
        return carry

    lax.fori_loop(0, P_NODES // 16, chunk_step, 0)
    pltpu.sync_copy(out_v, out_hbm.at[pl.ds(base * OUT_CH,
                                            P_NODES * OUT_CH)])


def _final_call(partials_flat, root_flat):
    k = functools.partial(
        pl.kernel,
        out_type=jax.ShapeDtypeStruct((NPAD * OUT_CH,), jnp.float32),
        mesh=_sc_mesh(),
        compiler_params=pltpu.CompilerParams(needs_layout_passes=False),
        scratch_types=[
            pltpu.VMEM((NSPLIT, OUT_CH, P_NODES), jnp.float32),
            pltpu.VMEM((P_NODES * OUT_CH,), jnp.float32),
            pltpu.VMEM((P_NODES * OUT_CH,), jnp.float32),
        ],
    )(_final_body)
    return k(partials_flat.reshape(NSPLIT, OUT_CH, NPAD), root_flat)


def kernel(node_feats, edge_index, edge_feats, W1, b1, W2, b2, W3, b3, root,
           bias):
    src = edge_index[0].astype(jnp.int32)
    dst = edge_index[1].astype(jnp.int32)
    W3q = (W3.reshape(HIDDEN, NODE_IN, OUT_CH)
           .transpose(1, 2, 0)
           .reshape(NODE_IN, OUT_CH * HIDDEN)).astype(jnp.bfloat16)
    S2 = jnp.repeat(jnp.eye(OUT_CH, dtype=jnp.bfloat16), HIDDEN, axis=0)
    B3r = b3.reshape(NODE_IN, OUT_CH).astype(jnp.bfloat16)
    b1r = b1.reshape(1, HIDDEN)
    b2r = b2.reshape(1, HIDDEN)
    biasr = bias.reshape(1, OUT_CH)

    xj1 = _gather_call(node_feats, src, 0, E_PH1)
    msg1 = _msg_call(edge_feats, xj1, W1, b1r, W2, b2r, W3q, S2, B3r,
                     0, E_PH1)
    xj2 = _gather_call(node_feats, src, E_PH1, E_PH2)
    msg2 = _msg_call(edge_feats, xj2, W1, b1r, W2, b2r, W3q, S2, B3r,
                     E_PH1 // BLK_E, E_PH2)
    root_term = _root_call(node_feats, root, biasr)
    p1 = _scatter_call(msg1.reshape(-1), dst, 0, E_PH1)
    p2 = _scatter_call(msg2.reshape(-1), dst, E_PH1, E_PH2, part_in=p1)
    out_pad = _final_call(p2, root_term.reshape(-1))
    return out_pad.reshape(NPAD, OUT_CH)[:N_NODES]
